# baseline (speedup 1.0000x reference)
import functools

import jax
import jax.numpy as jnp
from jax.experimental import pallas as pl
from jax.experimental.pallas import tpu as pltpu

_EPS = 1e-5
_CHUNK = 1024
_K = 5


def _body(x_hbm, m_ref, pos_hbm, seg_ref, g_ref, b_ref, o_hbm,
          in_buf, out_buf, pos_buf, in_sems, out_sems, pos_sems,
          *, n_chunks, n_pos_chunks):
    i = pl.program_id(0)
    slot = jax.lax.rem(i, _K)

    def in_copy(c, s):
        return pltpu.make_async_copy(
            x_hbm.at[pl.ds(c * _CHUNK, _CHUNK), :],
            in_buf.at[s], in_sems.at[s])

    def out_copy(c, s):
        return pltpu.make_async_copy(
            out_buf.at[s],
            o_hbm.at[pl.ds(c * _CHUNK, _CHUNK), :], out_sems.at[s])

    @pl.when(i == 0)
    def _():
        for j in range(n_pos_chunks):
            pltpu.make_async_copy(
                pos_hbm.at[pl.ds(j * _CHUNK, _CHUNK), :],
                pos_buf.at[pl.ds(j * _CHUNK, _CHUNK), :],
                pos_sems.at[j]).start()
        for j in range(_K):
            in_copy(j, j).start()

    @pl.when(i < n_pos_chunks)
    def _():
        pltpu.make_async_copy(
            pos_hbm.at[pl.ds(0, _CHUNK), :],
            pos_buf.at[pl.ds(0, _CHUNK), :],
            pos_sems.at[jax.lax.rem(i, n_pos_chunks)]).wait()

    in_copy(i, slot).wait()

    @pl.when(i >= _K)
    def _():
        out_copy(i - _K, slot).wait()

    pos_off = jax.lax.rem(i, n_pos_chunks) * _CHUNK
    m = m_ref[...] != 0
    seg = seg_ref[...]
    d_inv = 1.0 / seg.shape[-1]
    e = (in_buf[slot] + pos_buf[pl.ds(pos_off, _CHUNK), :]
         + jnp.where(m, seg[1][None, :], seg[0][None, :]))
    out_buf[slot] = e
    s1 = jnp.sum(e, axis=-1, keepdims=True)
    s2 = jnp.sum(e * e, axis=-1, keepdims=True)
    mu = s1 * d_inv
    var = s2 * d_inv - mu * mu
    r = jax.lax.rsqrt(var + _EPS)
    e2 = out_buf[slot]
    out_buf[slot] = (e2 - mu) * r * g_ref[...] + b_ref[...]

    out_copy(i, slot).start()

    @pl.when(i + _K < n_chunks)
    def _():
        in_copy(i + _K, slot).start()

    @pl.when(i == n_chunks - 1)
    def _():
        for t in range(_K):
            c = n_chunks - _K + t
            out_copy(c, c % _K).wait()


@functools.partial(jax.jit, static_argnames=("interpret",))
def _run(x, maskb, pos_emb_w, seg_emb_w, gamma, beta, interpret=False):
    B, S, D = x.shape
    n_chunks = (B * S) // _CHUNK
    n_pos_chunks = S // _CHUNK
    xf = x.reshape(B * S, D)
    mf = maskb.reshape(B * S, 1)

    out = pl.pallas_call(
        functools.partial(_body, n_chunks=n_chunks, n_pos_chunks=n_pos_chunks),
        grid=(n_chunks,),
        in_specs=[
            pl.BlockSpec(memory_space=pl.ANY),
            pl.BlockSpec((_CHUNK, 1), lambda i: (i, 0)),
            pl.BlockSpec(memory_space=pl.ANY),
            pl.BlockSpec((2, D), lambda i: (0, 0)),
            pl.BlockSpec((1, D), lambda i: (0, 0)),
            pl.BlockSpec((1, D), lambda i: (0, 0)),
        ],
        out_specs=pl.BlockSpec(memory_space=pl.ANY),
        out_shape=jax.ShapeDtypeStruct((B * S, D), x.dtype),
        scratch_shapes=[
            pltpu.VMEM((_K, _CHUNK, D), jnp.float32),
            pltpu.VMEM((_K, _CHUNK, D), jnp.float32),
            pltpu.VMEM((S, D), jnp.float32),
            pltpu.SemaphoreType.DMA((_K,)),
            pltpu.SemaphoreType.DMA((_K,)),
            pltpu.SemaphoreType.DMA((S // _CHUNK,)),
        ],
        compiler_params=pltpu.CompilerParams(
            dimension_semantics=("arbitrary",),
            vmem_limit_bytes=128 * 1024 * 1024),
        interpret=interpret,
    )(xf, mf, pos_emb_w, seg_emb_w, gamma.reshape(1, D), beta.reshape(1, D))
    return out.reshape(B, S, D)


def kernel(x, segment_mask, pos_emb_w, seg_emb_w, gamma, beta):
    maskb = segment_mask.astype(jnp.int8)
    return _run(x, maskb, pos_emb_w, seg_emb_w, gamma, beta)


# speedup vs baseline: 1.0951x; 1.0951x over previous
import functools

import jax
import jax.numpy as jnp
from jax.experimental import pallas as pl
from jax.experimental.pallas import tpu as pltpu

_EPS = 1e-5
_CHUNK = 1024
_K = 4


def _body(x_hbm, m_hbm, pos_hbm, seg_ref, g_ref, b_ref, o_hbm,
          in_buf, out_buf, pos_buf, m_buf, in_sems, out_sems, pos_sems, m_sem,
          *, n_chunks, n_pos_chunks):
    i = pl.program_id(0)
    slot = jax.lax.rem(i, _K)

    def in_copy(c, s):
        return pltpu.make_async_copy(
            x_hbm.at[pl.ds(c * _CHUNK, _CHUNK), :],
            in_buf.at[s], in_sems.at[s])

    def out_copy(c, s):
        return pltpu.make_async_copy(
            out_buf.at[s],
            o_hbm.at[pl.ds(c * _CHUNK, _CHUNK), :], out_sems.at[s])

    @pl.when(i == 0)
    def _():
        m_cp = pltpu.make_async_copy(m_hbm, m_buf, m_sem)
        m_cp.start()
        for j in range(n_pos_chunks):
            pltpu.make_async_copy(
                pos_hbm.at[pl.ds(j * _CHUNK, _CHUNK), :],
                pos_buf.at[pl.ds(j * _CHUNK, _CHUNK), :],
                pos_sems.at[j]).start()
        for j in range(_K):
            in_copy(j, j).start()
        m_cp.wait()

    @pl.when(i < n_pos_chunks)
    def _():
        pltpu.make_async_copy(
            pos_hbm.at[pl.ds(0, _CHUNK), :],
            pos_buf.at[pl.ds(0, _CHUNK), :],
            pos_sems.at[jax.lax.rem(i, n_pos_chunks)]).wait()

    in_copy(i, slot).wait()

    @pl.when(i >= _K)
    def _():
        out_copy(i - _K, slot).wait()

    pos_off = jax.lax.rem(i, n_pos_chunks) * _CHUNK
    m = m_buf[pl.ds(i * _CHUNK, _CHUNK), :] != 0
    seg = seg_ref[...]
    d_inv = 1.0 / seg.shape[-1]
    e = (in_buf[slot] + pos_buf[pl.ds(pos_off, _CHUNK), :]
         + jnp.where(m, seg[1][None, :], seg[0][None, :]))
    out_buf[slot] = e
    s1 = jnp.sum(e, axis=-1, keepdims=True)
    s2 = jnp.sum(e * e, axis=-1, keepdims=True)
    mu = s1 * d_inv
    var = s2 * d_inv - mu * mu
    r = jax.lax.rsqrt(var + _EPS)
    e2 = out_buf[slot]
    out_buf[slot] = (e2 - mu) * r * g_ref[...] + b_ref[...]

    out_copy(i, slot).start()

    @pl.when(i + _K < n_chunks)
    def _():
        in_copy(i + _K, slot).start()

    @pl.when(i == n_chunks - 1)
    def _():
        for t in range(_K):
            c = n_chunks - _K + t
            out_copy(c, c % _K).wait()


@functools.partial(jax.jit, static_argnames=("interpret",))
def _run(x, maskb, pos_emb_w, seg_emb_w, gamma, beta, interpret=False):
    B, S, D = x.shape
    n_chunks = (B * S) // _CHUNK
    n_pos_chunks = S // _CHUNK
    xf = x.reshape(B * S, D)
    mf = maskb.reshape(B * S, 1)

    out = pl.pallas_call(
        functools.partial(_body, n_chunks=n_chunks, n_pos_chunks=n_pos_chunks),
        grid=(n_chunks,),
        in_specs=[
            pl.BlockSpec(memory_space=pl.ANY),
            pl.BlockSpec(memory_space=pl.ANY),
            pl.BlockSpec(memory_space=pl.ANY),
            pl.BlockSpec((2, D), lambda i: (0, 0)),
            pl.BlockSpec((1, D), lambda i: (0, 0)),
            pl.BlockSpec((1, D), lambda i: (0, 0)),
        ],
        out_specs=pl.BlockSpec(memory_space=pl.ANY),
        out_shape=jax.ShapeDtypeStruct((B * S, D), x.dtype),
        scratch_shapes=[
            pltpu.VMEM((_K, _CHUNK, D), jnp.float32),
            pltpu.VMEM((_K, _CHUNK, D), jnp.float32),
            pltpu.VMEM((S, D), jnp.float32),
            pltpu.VMEM((B * S, 1), jnp.int8),
            pltpu.SemaphoreType.DMA((_K,)),
            pltpu.SemaphoreType.DMA((_K,)),
            pltpu.SemaphoreType.DMA((S // _CHUNK,)),
            pltpu.SemaphoreType.DMA,
        ],
        compiler_params=pltpu.CompilerParams(
            dimension_semantics=("arbitrary",),
            vmem_limit_bytes=128 * 1024 * 1024),
        interpret=interpret,
    )(xf, mf, pos_emb_w, seg_emb_w, gamma.reshape(1, D), beta.reshape(1, D))
    return out.reshape(B, S, D)


def kernel(x, segment_mask, pos_emb_w, seg_emb_w, gamma, beta):
    maskb = segment_mask.astype(jnp.int8)
    return _run(x, maskb, pos_emb_w, seg_emb_w, gamma, beta)


# speedup vs baseline: 1.1125x; 1.0159x over previous
import functools

import jax
import jax.numpy as jnp
from jax.experimental import pallas as pl
from jax.experimental.pallas import tpu as pltpu

_EPS = 1e-5
_CHUNK = 1024
_K = 4


def _body(x_hbm, m_hbm, pos_hbm, seg_hbm, g_hbm, b_hbm, o_hbm,
          in_buf, out_buf, pos_buf, m_buf, sgb_buf,
          in_sems, out_sems, pos_sems, m_sem, sgb_sems,
          *, n_chunks, n_pos_chunks):
    i = pl.program_id(0)
    slot = jax.lax.rem(i, _K)

    def in_copy(c, s):
        return pltpu.make_async_copy(
            x_hbm.at[pl.ds(c * _CHUNK, _CHUNK), :],
            in_buf.at[s], in_sems.at[s])

    def out_copy(c, s):
        return pltpu.make_async_copy(
            out_buf.at[s],
            o_hbm.at[pl.ds(c * _CHUNK, _CHUNK), :], out_sems.at[s])

    @pl.when(i == 0)
    def _():
        m_cp = pltpu.make_async_copy(m_hbm, m_buf, m_sem)
        m_cp.start()
        s_cp = pltpu.make_async_copy(seg_hbm, sgb_buf.at[0:2], sgb_sems.at[0])
        g_cp = pltpu.make_async_copy(g_hbm, sgb_buf.at[2:3], sgb_sems.at[1])
        b_cp = pltpu.make_async_copy(b_hbm, sgb_buf.at[3:4], sgb_sems.at[2])
        s_cp.start(); g_cp.start(); b_cp.start()
        for j in range(n_pos_chunks):
            pltpu.make_async_copy(
                pos_hbm.at[pl.ds(j * _CHUNK, _CHUNK), :],
                pos_buf.at[pl.ds(j * _CHUNK, _CHUNK), :],
                pos_sems.at[j]).start()
        for j in range(_K):
            in_copy(j, j).start()
        m_cp.wait(); s_cp.wait(); g_cp.wait(); b_cp.wait()

    @pl.when(i < n_pos_chunks)
    def _():
        pltpu.make_async_copy(
            pos_hbm.at[pl.ds(0, _CHUNK), :],
            pos_buf.at[pl.ds(0, _CHUNK), :],
            pos_sems.at[jax.lax.rem(i, n_pos_chunks)]).wait()

    in_copy(i, slot).wait()

    @pl.when(i >= _K)
    def _():
        out_copy(i - _K, slot).wait()

    pos_off = jax.lax.rem(i, n_pos_chunks) * _CHUNK
    m = m_buf[pl.ds(i * _CHUNK, _CHUNK), :] != 0
    seg = sgb_buf[0:2]
    d_inv = 1.0 / seg.shape[-1]
    e = (in_buf[slot] + pos_buf[pl.ds(pos_off, _CHUNK), :]
         + jnp.where(m, seg[1][None, :], seg[0][None, :]))
    out_buf[slot] = e
    s1 = jnp.sum(e, axis=-1, keepdims=True)
    s2 = jnp.sum(e * e, axis=-1, keepdims=True)
    mu = s1 * d_inv
    var = s2 * d_inv - mu * mu
    r = jax.lax.rsqrt(var + _EPS)
    e2 = out_buf[slot]
    out_buf[slot] = (e2 - mu) * r * sgb_buf[2:3] + sgb_buf[3:4]

    out_copy(i, slot).start()

    @pl.when(i + _K < n_chunks)
    def _():
        in_copy(i + _K, slot).start()

    @pl.when(i == n_chunks - 1)
    def _():
        for t in range(_K):
            c = n_chunks - _K + t
            out_copy(c, c % _K).wait()


@functools.partial(jax.jit, static_argnames=("interpret",))
def _run(x, maskb, pos_emb_w, seg_emb_w, gamma, beta, interpret=False):
    B, S, D = x.shape
    n_chunks = (B * S) // _CHUNK
    n_pos_chunks = S // _CHUNK
    xf = x.reshape(B * S, D)
    mf = maskb.reshape(B * S, 1)

    out = pl.pallas_call(
        functools.partial(_body, n_chunks=n_chunks, n_pos_chunks=n_pos_chunks),
        grid=(n_chunks,),
        in_specs=[
            pl.BlockSpec(memory_space=pl.ANY),
            pl.BlockSpec(memory_space=pl.ANY),
            pl.BlockSpec(memory_space=pl.ANY),
            pl.BlockSpec(memory_space=pl.ANY),
            pl.BlockSpec(memory_space=pl.ANY),
            pl.BlockSpec(memory_space=pl.ANY),
        ],
        out_specs=pl.BlockSpec(memory_space=pl.ANY),
        out_shape=jax.ShapeDtypeStruct((B * S, D), x.dtype),
        scratch_shapes=[
            pltpu.VMEM((_K, _CHUNK, D), jnp.float32),
            pltpu.VMEM((_K, _CHUNK, D), jnp.float32),
            pltpu.VMEM((S, D), jnp.float32),
            pltpu.VMEM((B * S, 1), jnp.int8),
            pltpu.VMEM((4, D), jnp.float32),
            pltpu.SemaphoreType.DMA((_K,)),
            pltpu.SemaphoreType.DMA((_K,)),
            pltpu.SemaphoreType.DMA((S // _CHUNK,)),
            pltpu.SemaphoreType.DMA,
            pltpu.SemaphoreType.DMA((3,)),
        ],
        compiler_params=pltpu.CompilerParams(
            dimension_semantics=("arbitrary",),
            vmem_limit_bytes=128 * 1024 * 1024),
        interpret=interpret,
    )(xf, mf, pos_emb_w, seg_emb_w, gamma.reshape(1, D), beta.reshape(1, D))
    return out.reshape(B, S, D)


def kernel(x, segment_mask, pos_emb_w, seg_emb_w, gamma, beta):
    maskb = segment_mask.astype(jnp.int8)
    return _run(x, maskb, pos_emb_w, seg_emb_w, gamma, beta)
